# baseline (speedup 1.0000x reference)
import jax
import jax.numpy as jnp
from jax.experimental import pallas as pl
from jax.experimental.pallas import tpu as pltpu

_CR = 8
_NBUF = 8


def _bias_add_dma_kernel(x_hbm, b_vmem, o_hbm, in_buf, out_buf, in_sem, out_sem):
    n_chunks = x_hbm.shape[0] // _CR

    def start_in(chunk, buf):
        pltpu.make_async_copy(
            x_hbm.at[pl.ds(chunk * _CR, _CR), :], in_buf.at[buf], in_sem.at[buf]
        ).start()

    def start_out(chunk, buf):
        pltpu.make_async_copy(
            out_buf.at[buf], o_hbm.at[pl.ds(chunk * _CR, _CR), :], out_sem.at[buf]
        ).start()

    for s in range(_NBUF):
        start_in(s, s)

    def body(i, _):
        b = jax.lax.rem(i, _NBUF)
        pltpu.make_async_copy(
            x_hbm.at[pl.ds(i * _CR, _CR), :], in_buf.at[b], in_sem.at[b]
        ).wait()

        @pl.when(i >= _NBUF)
        def _():
            pltpu.make_async_copy(
                out_buf.at[b], o_hbm.at[pl.ds((i - _NBUF) * _CR, _CR), :], out_sem.at[b]
            ).wait()

        out_buf[b] = in_buf[b] + b_vmem[...]
        start_out(i, b)

        @pl.when(i + _NBUF < n_chunks)
        def _():
            start_in(i + _NBUF, b)

        return 0

    jax.lax.fori_loop(0, n_chunks, body, 0)

    def drain(i, _):
        b = jax.lax.rem(i, _NBUF)
        pltpu.make_async_copy(
            out_buf.at[b], o_hbm.at[pl.ds(i * _CR, _CR), :], out_sem.at[b]
        ).wait()
        return 0

    jax.lax.fori_loop(n_chunks - _NBUF, n_chunks, drain, 0)


def kernel(user_emb, loc_logits, user_loc_weights, loc_bias):
    B, L = loc_logits.shape
    bias2d = loc_bias.reshape(1, L)
    out = pl.pallas_call(
        _bias_add_dma_kernel,
        in_specs=[
            pl.BlockSpec(memory_space=pltpu.MemorySpace.HBM),
            pl.BlockSpec(memory_space=pltpu.VMEM),
        ],
        out_specs=pl.BlockSpec(memory_space=pltpu.MemorySpace.HBM),
        out_shape=jax.ShapeDtypeStruct((B, L), jnp.float32),
        scratch_shapes=[
            pltpu.VMEM((_NBUF, _CR, L), jnp.float32),
            pltpu.VMEM((_NBUF, _CR, L), jnp.float32),
            pltpu.SemaphoreType.DMA((_NBUF,)),
            pltpu.SemaphoreType.DMA((_NBUF,)),
        ],
        compiler_params=pltpu.CompilerParams(vmem_limit_bytes=60 * 1024 * 1024),
    )(loc_logits, bias2d)
    return out
